# Initial kernel scaffold; baseline (speedup 1.0000x reference)
import jax
import jax.numpy as jnp
from jax.experimental import pallas as pl


def kernel(x, edge_index, batch, g0_w1, g0_b1, g0_w2, g0_b2, g1_w1, g1_b1, g1_w2, g1_b2, g2_w1, g2_b1, g2_w2, g2_b2, l0_w, l0_b, l1_w, l1_b, out_w, out_b):
    raise NotImplementedError("write your pallas kernel here")


# speedup vs baseline: 4.7676x; 4.7676x over previous
import functools

import jax
import jax.numpy as jnp
from jax import lax
from jax.experimental import pallas as pl
from jax.experimental.pallas import tpu as pltpu
from jax.experimental.pallas import tpu_sc as plsc

NC = 2
NS = 16
CHUNK = 128


def _seg_sum_sc(y, src_p, dst_p, n_pad, cpt):
    h = y.shape[1]
    rows_per_tile = n_pad // NS

    @functools.partial(
        pl.kernel,
        out_type=jax.ShapeDtypeStruct((NC, n_pad, h), jnp.float32),
        mesh=plsc.VectorSubcoreMesh(
            core_axis_name="c", subcore_axis_name="s",
            num_cores=NC, num_subcores=NS),
        scratch_types=[
            pltpu.VMEM((CHUNK,), jnp.int32),
            pltpu.VMEM((CHUNK,), jnp.int32),
            pltpu.VMEM((CHUNK, h), jnp.float32),
            pltpu.VMEM_SHARED((n_pad, h), jnp.float32),
            pltpu.SemaphoreType.DMA,
        ],
        compiler_params=pltpu.CompilerParams(use_tc_tiling_on_sc=False),
    )
    def agg(y_hbm, src_hbm, dst_hbm, zero_hbm, out_hbm,
            src_v, dst_v, rows_v, acc_sh, sem):
        cid = lax.axis_index("c")
        sid = lax.axis_index("s")
        wid = cid * NS + sid

        r0 = sid * rows_per_tile
        pltpu.sync_copy(zero_hbm, rows_v)
        for k in range(rows_per_tile // CHUNK):
            pltpu.sync_copy(rows_v, acc_sh.at[pl.ds(r0 + k * CHUNK, CHUNK)])
        plsc.subcore_barrier()

        def body(j, carry):
            base = pl.multiple_of((wid * cpt + j) * CHUNK, CHUNK)
            pltpu.sync_copy(src_hbm.at[pl.ds(base, CHUNK)], src_v)
            pltpu.sync_copy(dst_hbm.at[pl.ds(base, CHUNK)], dst_v)
            pltpu.async_copy(y_hbm.at[src_v], rows_v, sem).wait()
            pltpu.sync_copy(rows_v, acc_sh.at[dst_v], add=True)
            return carry

        lax.fori_loop(0, cpt, body, 0)
        plsc.subcore_barrier()

        for k in range(rows_per_tile // CHUNK):
            pltpu.sync_copy(acc_sh.at[pl.ds(r0 + k * CHUNK, CHUNK)], rows_v)
            pltpu.sync_copy(
                rows_v, out_hbm.at[cid, pl.ds(r0 + k * CHUNK, CHUNK)])

    zero = jnp.zeros((CHUNK, h), jnp.float32)
    return agg(y, src_p, dst_p, zero)


def _proj_tc(x, w, blk):
    n, d = x.shape
    h = w.shape[1]

    def body(x_ref, w_ref, o_ref):
        o_ref[...] = jnp.dot(x_ref[...], w_ref[...],
                             preferred_element_type=jnp.float32)

    return pl.pallas_call(
        body,
        grid=(n // blk,),
        in_specs=[pl.BlockSpec((blk, d), lambda i: (i, 0)),
                  pl.BlockSpec((d, h), lambda i: (0, 0))],
        out_specs=pl.BlockSpec((blk, h), lambda i: (i, 0)),
        out_shape=jax.ShapeDtypeStruct((n, h), jnp.float32),
    )(x, w)


def _mid_tc(y, aggs, b1, w2, b2, w1n, blk):
    n, h = y.shape

    def body(y_ref, a_ref, b1_ref, w2_ref, b2_ref, w1n_ref, o_ref):
        t = jnp.maximum(y_ref[...] + a_ref[0] + a_ref[1] + b1_ref[...], 0.0)
        hh = jnp.dot(t, w2_ref[...],
                     preferred_element_type=jnp.float32) + b2_ref[...]
        xn = jnp.maximum(hh, 0.0)
        o_ref[...] = jnp.dot(xn, w1n_ref[...],
                             preferred_element_type=jnp.float32)

    return pl.pallas_call(
        body,
        grid=(n // blk,),
        in_specs=[pl.BlockSpec((blk, h), lambda i: (i, 0)),
                  pl.BlockSpec((NC, blk, h), lambda i: (0, i, 0)),
                  pl.BlockSpec((1, h), lambda i: (0, 0)),
                  pl.BlockSpec((h, h), lambda i: (0, 0)),
                  pl.BlockSpec((1, h), lambda i: (0, 0)),
                  pl.BlockSpec((h, h), lambda i: (0, 0))],
        out_specs=pl.BlockSpec((blk, h), lambda i: (i, 0)),
        out_shape=jax.ShapeDtypeStruct((n, h), jnp.float32),
    )(y, aggs, b1, w2, b2, w1n)


def _final_tc(y, aggs, b1, w2, b2, batch_f, g,
              l0_w, l0_b, l1_w, l1_b, out_w, out_b, blk):
    n, h = y.shape

    def body(y_ref, a_ref, b1_ref, w2_ref, b2_ref, bat_ref,
             l0w_ref, l0b_ref, l1w_ref, l1b_ref, ow_ref, ob_ref,
             o_ref, sums_ref, cnts_ref):
        i = pl.program_id(0)
        t = jnp.maximum(y_ref[...] + a_ref[0] + a_ref[1] + b1_ref[...], 0.0)
        hh = jnp.dot(t, w2_ref[...],
                     preferred_element_type=jnp.float32) + b2_ref[...]
        gid = lax.broadcasted_iota(jnp.int32, (blk, g), 1)
        onehot = jnp.where(bat_ref[...] == gid, 1.0, 0.0)
        ps = lax.dot_general(onehot, hh, (((0,), (0,)), ((), ())),
                             preferred_element_type=jnp.float32)
        pc = lax.dot_general(onehot, jnp.ones_like(hh),
                             (((0,), (0,)), ((), ())),
                             preferred_element_type=jnp.float32)

        @pl.when(i == 0)
        def _():
            sums_ref[...] = jnp.zeros_like(sums_ref)
            cnts_ref[...] = jnp.zeros_like(cnts_ref)

        sums_ref[...] += ps
        cnts_ref[...] += pc

        @pl.when(i == pl.num_programs(0) - 1)
        def _():
            pooled = sums_ref[...] / jnp.maximum(cnts_ref[...], 1.0)
            z = jnp.maximum(pooled, 0.0)
            z = jnp.maximum(
                jnp.dot(z, l0w_ref[...],
                        preferred_element_type=jnp.float32) + l0b_ref[...],
                0.0)
            z = jnp.maximum(
                jnp.dot(z, l1w_ref[...],
                        preferred_element_type=jnp.float32) + l1b_ref[...],
                0.0)
            o_ref[...] = jnp.dot(
                z, ow_ref[...],
                preferred_element_type=jnp.float32) + ob_ref[...]

    return pl.pallas_call(
        body,
        grid=(n // blk,),
        in_specs=[pl.BlockSpec((blk, h), lambda i: (i, 0)),
                  pl.BlockSpec((NC, blk, h), lambda i: (0, i, 0)),
                  pl.BlockSpec((1, h), lambda i: (0, 0)),
                  pl.BlockSpec((h, h), lambda i: (0, 0)),
                  pl.BlockSpec((1, h), lambda i: (0, 0)),
                  pl.BlockSpec((blk, 1), lambda i: (i, 0)),
                  pl.BlockSpec((h, h), lambda i: (0, 0)),
                  pl.BlockSpec((1, h), lambda i: (0, 0)),
                  pl.BlockSpec((h, h), lambda i: (0, 0)),
                  pl.BlockSpec((1, h), lambda i: (0, 0)),
                  pl.BlockSpec((h, 1), lambda i: (0, 0)),
                  pl.BlockSpec((1, 1), lambda i: (0, 0))],
        out_specs=pl.BlockSpec((g, 1), lambda i: (0, 0)),
        out_shape=jax.ShapeDtypeStruct((g, 1), jnp.float32),
        scratch_shapes=[pltpu.VMEM((g, h), jnp.float32),
                        pltpu.VMEM((g, h), jnp.float32)],
    )(y, aggs, b1, w2, b2, batch_f,
      l0_w, l0_b, l1_w, l1_b, out_w, out_b)


def kernel(x, edge_index, batch,
           g0_w1, g0_b1, g0_w2, g0_b2,
           g1_w1, g1_b1, g1_w2, g1_b2,
           g2_w1, g2_b1, g2_w2, g2_b2,
           l0_w, l0_b, l1_w, l1_b, out_w, out_b):
    n = x.shape[0]
    h = g0_w1.shape[1]
    g = 128
    e = edge_index.shape[1]
    blk = 2000

    n_pad = ((n + NS * CHUNK - 1) // (NS * CHUNK)) * (NS * CHUNK)
    per = NC * NS * CHUNK
    cpt = (e + per - 1) // per
    e_pad = cpt * per
    src_p = jnp.concatenate(
        [edge_index[0], jnp.zeros((e_pad - e,), jnp.int32)])
    dst_p = jnp.concatenate(
        [edge_index[1], jnp.full((e_pad - e,), n, jnp.int32)])
    batch_f = batch.reshape(n, 1)

    def row(b):
        return b.reshape(1, -1)

    y0 = _proj_tc(x, g0_w1, blk)
    a0 = _seg_sum_sc(y0, src_p, dst_p, n_pad, cpt)
    y1 = _mid_tc(y0, a0, row(g0_b1), g0_w2, row(g0_b2), g1_w1, blk)
    a1 = _seg_sum_sc(y1, src_p, dst_p, n_pad, cpt)
    y2 = _mid_tc(y1, a1, row(g1_b1), g1_w2, row(g1_b2), g2_w1, blk)
    a2 = _seg_sum_sc(y2, src_p, dst_p, n_pad, cpt)
    z = _final_tc(y2, a2, row(g2_b1), g2_w2, row(g2_b2), batch_f, g,
                  l0_w, row(l0_b), l1_w, row(l1_b),
                  out_w, out_b.reshape(1, 1), blk)
    return z.reshape(-1)
